# baseline (speedup 1.0000x reference)
import jax
import jax.numpy as jnp
from jax import lax
from jax.experimental import pallas as pl
from jax.experimental.pallas import tpu as pltpu
from jax.experimental.pallas import tpu_sc as plsc

_N_NODES = 10000
_D = 128
_DH = _D // 2
_M = 320000
_S = 5000
_NC, _NS = 2, 16
_S_PAD = 5120
_ROWS_PER_TILE = _S_PAD // _NS
_PER_T = _M // _NS
_C = 80
_NCHUNK = _PER_T // _C
_K = 5
_NGRP = _NCHUNK // _K
_L = 16


def _sc_body(tbl_cat, nid2, seg_ids3, zrow,
             out, *scratch):
    idx_n, idx_s, cnt_loc, cvm, ivm = scratch[:5]
    rows = scratch[5:5 + 2 * _K]
    sums_sp, counts_sp = scratch[5 + 2 * _K:7 + 2 * _K]
    gsem = scratch[7 + 2 * _K:9 + 2 * _K]
    ssem = scratch[9 + 2 * _K:11 + 2 * _K]

    cid = lax.axis_index("c")
    sid = lax.axis_index("s")
    base = sid * _PER_T
    r0 = sid * _ROWS_PER_TILE

    a = pltpu.async_copy(nid2.at[cid, pl.ds(base, _PER_T)], idx_n, gsem[0])
    b = pltpu.async_copy(seg_ids3.at[sid], idx_s, gsem[1])
    d = pltpu.async_copy(zrow, rows[0], ssem[1])
    zvec = jnp.zeros((_L,), jnp.float32)

    def zero_cnt(k, carry):
        cnt_loc[pl.ds(k * _L, _L)] = zvec
        return carry

    lax.fori_loop(0, _S_PAD // _L, zero_cnt, 0)
    a.wait()
    b.wait()
    d.wait()
    for j in range(_ROWS_PER_TILE // _C):
        pltpu.sync_copy(rows[0], sums_sp.at[pl.ds(r0 + j * _C, _C)])
    plsc.subcore_barrier()

    ones_vec = jnp.ones((_L,), jnp.float32)

    def issue_gather(i, buf, sem):
        pltpu.async_copy(tbl_cat.at[idx_n.at[pl.ds(i * _C, _C)]], buf, sem)

    for j in range(_K):
        issue_gather(j, rows[j], gsem[0])

    def super_body(u, carry):
        for p in (0, 1):
            t = 2 * u + p
            bb = p * _K
            nbb = (1 - p) * _K
            for j in range(_K):
                pltpu.make_async_copy(zrow, rows[bb + j], gsem[p]).wait()
            for j in range(_K):
                i = t * _K + j
                pltpu.async_copy(rows[bb + j], sums_sp.at[idx_s.at[i]],
                                 ssem[p], add=True)
                for m in range(_C // _L):
                    v = idx_s[i, pl.ds(m * _L, _L)]
                    plsc.addupdate_scatter(cnt_loc, [v], ones_vec)
            def drain_prev():
                for j in range(_K):
                    pltpu.make_async_copy(zrow, rows[nbb + j],
                                          ssem[1 - p]).wait()

            def issue_next():
                for j in range(_K):
                    issue_gather((t + 1) * _K + j, rows[nbb + j],
                                 gsem[1 - p])

            if p == 1:
                drain_prev()
                pl.when(u < (_NGRP // 2) - 1)(issue_next)
            else:
                pl.when(u >= 1)(drain_prev)
                issue_next()
        return carry

    lax.fori_loop(0, _NGRP // 2, super_body, 0)

    hist_pub = pltpu.async_copy(cnt_loc, counts_sp.at[sid], gsem[0])
    for j in range(_K):
        pltpu.make_async_copy(zrow, rows[_K + j], ssem[1]).wait()
    hist_pub.wait()
    plsc.subcore_barrier()

    cv = pltpu.async_copy(counts_sp.at[:, pl.ds(r0, _ROWS_PER_TILE)], cvm,
                          gsem[1])
    for j in range(_ROWS_PER_TILE // _C):
        pltpu.async_copy(sums_sp.at[pl.ds(r0 + j * _C, _C)], rows[j],
                         gsem[0])
    cv.wait()
    for g in range(_ROWS_PER_TILE // _L):
        acc = cvm[0, pl.ds(g * _L, _L)]
        for r in range(1, _NS):
            acc = acc + cvm[r, pl.ds(g * _L, _L)]
        ivm[pl.ds(g * _L, _L)] = 1.0 / jnp.maximum(acc, 1.0)

    for j in range(_ROWS_PER_TILE // _C):
        pltpu.make_async_copy(sums_sp.at[pl.ds(r0 + j * _C, _C)], rows[j],
                              gsem[0]).wait()

        def scale_row(r, carry):
            inv = plsc.load_gather(
                ivm, [jnp.full((_L,), j * _C, jnp.int32) + r])
            for m in range(_DH // _L):
                rows[j][r, pl.ds(m * _L, _L)] = (
                    rows[j][r, pl.ds(m * _L, _L)] * inv)
            return carry

        lax.fori_loop(0, _C, scale_row, 0)
        pltpu.async_copy(
            rows[j],
            out.at[pl.ds(r0 + j * _C, _C), pl.ds(cid * _DH, _DH)],
            ssem[0])
    for j in range(_ROWS_PER_TILE // _C):
        pltpu.make_async_copy(
            rows[j],
            out.at[pl.ds(r0 + j * _C, _C), pl.ds(cid * _DH, _DH)],
            ssem[0]).wait()


@jax.jit
def _impl(node_feature, batch_node_ids, batch_macro_node_ids):
    tbl_cat = jnp.concatenate(
        [node_feature[:, :_DH], node_feature[:, _DH:]], axis=0)
    nid2 = jnp.stack([batch_node_ids, batch_node_ids + _N_NODES])
    seg_ids3 = batch_macro_node_ids.reshape(_NS, _NCHUNK, _C)
    zrow = jnp.zeros((_C, _DH), jnp.float32)

    mesh = plsc.VectorSubcoreMesh(core_axis_name="c", subcore_axis_name="s")
    out_pad = pl.kernel(
        _sc_body,
        out_type=jax.ShapeDtypeStruct((_S_PAD, _D), jnp.float32),
        mesh=mesh,
        compiler_params=pltpu.CompilerParams(
            use_tc_tiling_on_sc=False, needs_layout_passes=False),
        scratch_types=[
            pltpu.VMEM((_PER_T,), jnp.int32),
            pltpu.VMEM((_NCHUNK, _C), jnp.int32),
            pltpu.VMEM((_S_PAD,), jnp.float32),
            pltpu.VMEM((_NS, _ROWS_PER_TILE), jnp.float32),
            pltpu.VMEM((_ROWS_PER_TILE,), jnp.float32),
        ] + [pltpu.VMEM((_C, _DH), jnp.float32) for _ in range(2 * _K)] + [
            pltpu.VMEM_SHARED((_S_PAD, _DH), jnp.float32),
            pltpu.VMEM_SHARED((_NS, _S_PAD), jnp.float32),
            pltpu.SemaphoreType.DMA,
            pltpu.SemaphoreType.DMA,
            pltpu.SemaphoreType.DMA,
            pltpu.SemaphoreType.DMA,
        ],
    )(tbl_cat, nid2, seg_ids3, zrow)

    return out_pad[:_S]


def kernel(node_feature, batch_node_ids, batch_macro_node_ids):
    return _impl(node_feature, batch_node_ids, batch_macro_node_ids)


# speedup vs baseline: 1.2407x; 1.2407x over previous
import jax
import jax.numpy as jnp
from jax import lax
from jax.experimental import pallas as pl
from jax.experimental.pallas import tpu as pltpu
from jax.experimental.pallas import tpu_sc as plsc

_N_NODES = 10000
_D = 128
_DH = _D // 2
_M = 320000
_S = 5000
_NC, _NS = 2, 16
_S_PAD = 5120
_ROWS_PER_TILE = _S_PAD // _NS
_PER_T = _M // _NS
_C = 80
_NCHUNK = _PER_T // _C
_K = 5
_NGRP = _NCHUNK // _K
_L = 16


def _sc_body(tbl2, node_ids, seg_ids3, out, *scratch):
    idx_n, idx_s, cnt_loc, cvm, ivm = scratch[:5]
    rows = scratch[5:5 + 2 * _K]
    sums_sp, counts_sp = scratch[5 + 2 * _K:7 + 2 * _K]
    gsem = scratch[7 + 2 * _K:9 + 2 * _K]
    ssem = scratch[9 + 2 * _K:11 + 2 * _K]

    cid = lax.axis_index("c")
    sid = lax.axis_index("s")
    base = sid * _PER_T
    r0 = sid * _ROWS_PER_TILE
    dz = tbl2.at[pl.ds(0, _C)]

    a = pltpu.async_copy(node_ids.at[pl.ds(base, _PER_T)], idx_n, gsem[0])
    b = pltpu.async_copy(seg_ids3.at[sid], idx_s, gsem[1])
    zvec = jnp.zeros((_L,), jnp.float32)

    def zero_cnt(k, carry):
        cnt_loc[pl.ds(k * _L, _L)] = zvec
        return carry

    lax.fori_loop(0, _S_PAD // _L, zero_cnt, 0)

    def zero_row(r, carry):
        for m in range(_DH // _L):
            rows[0][r, pl.ds(m * _L, _L)] = zvec
        return carry

    lax.fori_loop(0, _C, zero_row, 0)
    a.wait()
    b.wait()

    def xform(k, carry):
        v = idx_n[pl.ds(k * _L, _L)]
        idx_n[pl.ds(k * _L, _L)] = v + v + cid
        return carry

    lax.fori_loop(0, _K * _C // _L, xform, 0)

    def issue_gather(i, buf, sem):
        pltpu.async_copy(tbl2.at[idx_n.at[pl.ds(i * _C, _C)]], buf, sem)

    for j in range(_K):
        issue_gather(j, rows[j], gsem[0])

    lax.fori_loop(_K * _C // _L, _PER_T // _L, xform, 0)

    for j in range(_ROWS_PER_TILE // _C):
        pltpu.sync_copy(rows[0], sums_sp.at[pl.ds(r0 + j * _C, _C)])
    plsc.subcore_barrier()

    ones_vec = jnp.ones((_L,), jnp.float32)

    def super_body(u, carry):
        for p in (0, 1):
            t = 2 * u + p
            bb = p * _K
            nbb = (1 - p) * _K
            for j in range(_K):
                pltpu.make_async_copy(dz, rows[bb + j], gsem[p]).wait()
            for j in range(_K):
                i = t * _K + j
                pltpu.async_copy(rows[bb + j], sums_sp.at[idx_s.at[i]],
                                 ssem[p], add=True)
                for m in range(_C // _L):
                    v = idx_s[i, pl.ds(m * _L, _L)]
                    plsc.addupdate_scatter(cnt_loc, [v], ones_vec)
            def drain_prev():
                for j in range(_K):
                    pltpu.make_async_copy(dz, rows[nbb + j],
                                          ssem[1 - p]).wait()

            def issue_next():
                for j in range(_K):
                    issue_gather((t + 1) * _K + j, rows[nbb + j],
                                 gsem[1 - p])

            if p == 1:
                drain_prev()
                pl.when(u < (_NGRP // 2) - 1)(issue_next)
            else:
                pl.when(u >= 1)(drain_prev)
                issue_next()
        return carry

    lax.fori_loop(0, _NGRP // 2, super_body, 0)

    hist_pub = pltpu.async_copy(cnt_loc, counts_sp.at[sid], gsem[0])
    for j in range(_K):
        pltpu.make_async_copy(dz, rows[_K + j], ssem[1]).wait()
    hist_pub.wait()
    plsc.subcore_barrier()

    cv = pltpu.async_copy(counts_sp.at[:, pl.ds(r0, _ROWS_PER_TILE)], cvm,
                          gsem[1])
    for j in range(_ROWS_PER_TILE // _C):
        pltpu.async_copy(sums_sp.at[pl.ds(r0 + j * _C, _C)], rows[j],
                         gsem[0])
    cv.wait()
    for g in range(_ROWS_PER_TILE // _L):
        acc = cvm[0, pl.ds(g * _L, _L)]
        for r in range(1, _NS):
            acc = acc + cvm[r, pl.ds(g * _L, _L)]
        ivm[pl.ds(g * _L, _L)] = 1.0 / jnp.maximum(acc, 1.0)

    for j in range(_ROWS_PER_TILE // _C):
        pltpu.make_async_copy(sums_sp.at[pl.ds(r0 + j * _C, _C)], rows[j],
                              gsem[0]).wait()

        def scale_row(r, carry):
            inv = plsc.load_gather(
                ivm, [jnp.full((_L,), j * _C, jnp.int32) + r])
            for m in range(_DH // _L):
                rows[j][r, pl.ds(m * _L, _L)] = (
                    rows[j][r, pl.ds(m * _L, _L)] * inv)
            return carry

        lax.fori_loop(0, _C, scale_row, 0)

    def full_writes():
        for j in range(_ROWS_PER_TILE // _C):
            pltpu.async_copy(
                rows[j],
                out.at[pl.ds(r0 + j * _C, _C), pl.ds(cid * _DH, _DH)],
                ssem[0])
        for j in range(_ROWS_PER_TILE // _C):
            pltpu.make_async_copy(
                rows[j],
                out.at[pl.ds(r0 + j * _C, _C), pl.ds(cid * _DH, _DH)],
                ssem[0]).wait()

    def last_tile_writes():
        last = _S - (_NS - 1) * _ROWS_PER_TILE
        nfull = last // _C
        rem = last - nfull * _C
        for j in range(nfull):
            pltpu.async_copy(
                rows[j],
                out.at[pl.ds(r0 + j * _C, _C), pl.ds(cid * _DH, _DH)],
                ssem[0])
        pltpu.async_copy(
            rows[nfull].at[pl.ds(0, rem)],
            out.at[pl.ds(r0 + nfull * _C, rem), pl.ds(cid * _DH, _DH)],
            ssem[0])
        for j in range(nfull):
            pltpu.make_async_copy(
                rows[j],
                out.at[pl.ds(r0 + j * _C, _C), pl.ds(cid * _DH, _DH)],
                ssem[0]).wait()
        pltpu.make_async_copy(
            rows[nfull].at[pl.ds(0, rem)],
            out.at[pl.ds(r0 + nfull * _C, rem), pl.ds(cid * _DH, _DH)],
            ssem[0]).wait()

    pl.when(sid < _NS - 1)(full_writes)
    pl.when(sid == _NS - 1)(last_tile_writes)


@jax.jit
def _impl(node_feature, batch_node_ids, batch_macro_node_ids):
    tbl2 = node_feature.reshape(_N_NODES * _NC, _DH)
    seg_ids3 = batch_macro_node_ids.reshape(_NS, _NCHUNK, _C)

    mesh = plsc.VectorSubcoreMesh(core_axis_name="c", subcore_axis_name="s")
    out = pl.kernel(
        _sc_body,
        out_type=jax.ShapeDtypeStruct((_S, _D), jnp.float32),
        mesh=mesh,
        compiler_params=pltpu.CompilerParams(
            use_tc_tiling_on_sc=False, needs_layout_passes=False),
        scratch_types=[
            pltpu.VMEM((_PER_T,), jnp.int32),
            pltpu.VMEM((_NCHUNK, _C), jnp.int32),
            pltpu.VMEM((_S_PAD,), jnp.float32),
            pltpu.VMEM((_NS, _ROWS_PER_TILE), jnp.float32),
            pltpu.VMEM((_ROWS_PER_TILE,), jnp.float32),
        ] + [pltpu.VMEM((_C, _DH), jnp.float32) for _ in range(2 * _K)] + [
            pltpu.VMEM_SHARED((_S_PAD, _DH), jnp.float32),
            pltpu.VMEM_SHARED((_NS, _S_PAD), jnp.float32),
            pltpu.SemaphoreType.DMA,
            pltpu.SemaphoreType.DMA,
            pltpu.SemaphoreType.DMA,
            pltpu.SemaphoreType.DMA,
        ],
    )(tbl2, batch_node_ids, seg_ids3)

    return out


def kernel(node_feature, batch_node_ids, batch_macro_node_ids):
    return _impl(node_feature, batch_node_ids, batch_macro_node_ids)


# speedup vs baseline: 1.2412x; 1.0005x over previous
import jax
import jax.numpy as jnp
from jax import lax
from jax.experimental import pallas as pl
from jax.experimental.pallas import tpu as pltpu
from jax.experimental.pallas import tpu_sc as plsc

_N_NODES = 10000
_D = 128
_DH = _D // 2
_M = 320000
_S = 5000
_NC, _NS = 2, 16
_S_PAD = 5120
_ROWS_PER_TILE = _S_PAD // _NS
_PER_T = _M // _NS
_C = 80
_NCHUNK = _PER_T // _C
_K = 5
_NGRP = _NCHUNK // _K
_L = 16


def _sc_body(tbl2, node_ids, seg_ids3, out, *scratch):
    idx_n, idx_s, cnt_loc, cvm, ivm = scratch[:5]
    rows = scratch[5:5 + 2 * _K]
    sums_sp, counts_sp = scratch[5 + 2 * _K:7 + 2 * _K]
    gsem = scratch[7 + 2 * _K:9 + 2 * _K]
    ssem = scratch[9 + 2 * _K:11 + 2 * _K]

    cid = lax.axis_index("c")
    sid = lax.axis_index("s")
    base = sid * _PER_T
    r0 = sid * _ROWS_PER_TILE
    dz = tbl2.at[pl.ds(0, _C)]

    a = pltpu.async_copy(node_ids.at[pl.ds(base, _PER_T)], idx_n, gsem[0])
    b = pltpu.async_copy(seg_ids3.at[sid], idx_s, gsem[1])
    zvec = jnp.zeros((_L,), jnp.float32)

    def zero_cnt(k, carry):
        cnt_loc[pl.ds(k * _L, _L)] = zvec
        return carry

    lax.fori_loop(0, _S_PAD // _L, zero_cnt, 0)

    def zero_row(r, carry):
        for m in range(_DH // _L):
            rows[0][r, pl.ds(m * _L, _L)] = zvec
        return carry

    lax.fori_loop(0, _C, zero_row, 0)
    a.wait()
    b.wait()

    def xform(k, carry):
        v = idx_n[pl.ds(k * _L, _L)]
        idx_n[pl.ds(k * _L, _L)] = v + v + cid
        return carry

    lax.fori_loop(0, _K * _C // _L, xform, 0)

    for j in range(_ROWS_PER_TILE // _C):
        pltpu.sync_copy(rows[0], sums_sp.at[pl.ds(r0 + j * _C, _C)])

    def issue_gather(i, buf, sem):
        pltpu.async_copy(tbl2.at[idx_n.at[pl.ds(i * _C, _C)]], buf, sem)

    for j in range(_K):
        issue_gather(j, rows[j], gsem[0])

    lax.fori_loop(_K * _C // _L, _PER_T // _L, xform, 0)
    plsc.subcore_barrier()

    ones_vec = jnp.ones((_L,), jnp.float32)

    def super_body(u, carry):
        for p in (0, 1):
            t = 2 * u + p
            bb = p * _K
            nbb = (1 - p) * _K
            for j in range(_K):
                pltpu.make_async_copy(dz, rows[bb + j], gsem[p]).wait()
            for j in range(_K):
                i = t * _K + j
                pltpu.async_copy(rows[bb + j], sums_sp.at[idx_s.at[i]],
                                 ssem[p], add=True)
                for m in range(_C // _L):
                    v = idx_s[i, pl.ds(m * _L, _L)]
                    plsc.addupdate_scatter(cnt_loc, [v], ones_vec)
            def drain_prev():
                for j in range(_K):
                    pltpu.make_async_copy(dz, rows[nbb + j],
                                          ssem[1 - p]).wait()

            def issue_next():
                for j in range(_K):
                    issue_gather((t + 1) * _K + j, rows[nbb + j],
                                 gsem[1 - p])

            if p == 1:
                drain_prev()
                pl.when(u < (_NGRP // 2) - 1)(issue_next)
            else:
                pl.when(u >= 1)(drain_prev)
                issue_next()
        return carry

    lax.fori_loop(0, _NGRP // 2, super_body, 0)

    hist_pub = pltpu.async_copy(cnt_loc, counts_sp.at[sid], gsem[0])
    for j in range(_K):
        pltpu.make_async_copy(dz, rows[_K + j], ssem[1]).wait()
    hist_pub.wait()
    plsc.subcore_barrier()

    cv = pltpu.async_copy(counts_sp.at[:, pl.ds(r0, _ROWS_PER_TILE)], cvm,
                          gsem[1])
    for j in range(_ROWS_PER_TILE // _C):
        pltpu.async_copy(sums_sp.at[pl.ds(r0 + j * _C, _C)], rows[j],
                         gsem[0])
    cv.wait()
    for g in range(_ROWS_PER_TILE // _L):
        acc = cvm[0, pl.ds(g * _L, _L)]
        for r in range(1, _NS):
            acc = acc + cvm[r, pl.ds(g * _L, _L)]
        ivm[pl.ds(g * _L, _L)] = 1.0 / jnp.maximum(acc, 1.0)

    for j in range(_ROWS_PER_TILE // _C):
        pltpu.make_async_copy(sums_sp.at[pl.ds(r0 + j * _C, _C)], rows[j],
                              gsem[0]).wait()

        def scale_row(r, carry):
            inv = plsc.load_gather(
                ivm, [jnp.full((_L,), j * _C, jnp.int32) + r])
            for m in range(_DH // _L):
                rows[j][r, pl.ds(m * _L, _L)] = (
                    rows[j][r, pl.ds(m * _L, _L)] * inv)
            return carry

        lax.fori_loop(0, _C, scale_row, 0)

    def full_writes():
        for j in range(_ROWS_PER_TILE // _C):
            pltpu.async_copy(
                rows[j],
                out.at[pl.ds(r0 + j * _C, _C), pl.ds(cid * _DH, _DH)],
                ssem[0])
        for j in range(_ROWS_PER_TILE // _C):
            pltpu.make_async_copy(
                rows[j],
                out.at[pl.ds(r0 + j * _C, _C), pl.ds(cid * _DH, _DH)],
                ssem[0]).wait()

    def last_tile_writes():
        last = _S - (_NS - 1) * _ROWS_PER_TILE
        nfull = last // _C
        rem = last - nfull * _C
        for j in range(nfull):
            pltpu.async_copy(
                rows[j],
                out.at[pl.ds(r0 + j * _C, _C), pl.ds(cid * _DH, _DH)],
                ssem[0])
        pltpu.async_copy(
            rows[nfull].at[pl.ds(0, rem)],
            out.at[pl.ds(r0 + nfull * _C, rem), pl.ds(cid * _DH, _DH)],
            ssem[0])
        for j in range(nfull):
            pltpu.make_async_copy(
                rows[j],
                out.at[pl.ds(r0 + j * _C, _C), pl.ds(cid * _DH, _DH)],
                ssem[0]).wait()
        pltpu.make_async_copy(
            rows[nfull].at[pl.ds(0, rem)],
            out.at[pl.ds(r0 + nfull * _C, rem), pl.ds(cid * _DH, _DH)],
            ssem[0]).wait()

    pl.when(sid < _NS - 1)(full_writes)
    pl.when(sid == _NS - 1)(last_tile_writes)


@jax.jit
def _impl(node_feature, batch_node_ids, batch_macro_node_ids):
    tbl2 = node_feature.reshape(_N_NODES * _NC, _DH)
    seg_ids3 = batch_macro_node_ids.reshape(_NS, _NCHUNK, _C)

    mesh = plsc.VectorSubcoreMesh(core_axis_name="c", subcore_axis_name="s")
    out = pl.kernel(
        _sc_body,
        out_type=jax.ShapeDtypeStruct((_S, _D), jnp.float32),
        mesh=mesh,
        compiler_params=pltpu.CompilerParams(
            use_tc_tiling_on_sc=False, needs_layout_passes=False),
        scratch_types=[
            pltpu.VMEM((_PER_T,), jnp.int32),
            pltpu.VMEM((_NCHUNK, _C), jnp.int32),
            pltpu.VMEM((_S_PAD,), jnp.float32),
            pltpu.VMEM((_NS, _ROWS_PER_TILE), jnp.float32),
            pltpu.VMEM((_ROWS_PER_TILE,), jnp.float32),
        ] + [pltpu.VMEM((_C, _DH), jnp.float32) for _ in range(2 * _K)] + [
            pltpu.VMEM_SHARED((_S_PAD, _DH), jnp.float32),
            pltpu.VMEM_SHARED((_NS, _S_PAD), jnp.float32),
            pltpu.SemaphoreType.DMA,
            pltpu.SemaphoreType.DMA,
            pltpu.SemaphoreType.DMA,
            pltpu.SemaphoreType.DMA,
        ],
    )(tbl2, batch_node_ids, seg_ids3)

    return out


def kernel(node_feature, batch_node_ids, batch_macro_node_ids):
    return _impl(node_feature, batch_node_ids, batch_macro_node_ids)
